# baseline (speedup 1.0000x reference)
import functools

import jax
import jax.numpy as jnp
from jax import lax
from jax.experimental import pallas as pl
from jax.experimental.pallas import tpu as pltpu
from jax.experimental.pallas import tpu_sc as plsc

_HALF = 2048
_RADIX = 2048
_DIG_SHIFTS = (0, 11, 22)
_I32_MIN = -2147483648


def _sc_half_sort(x, m):
    nb, n, m_full = x.shape
    ncb = m // 16
    nchunks = nb * 2 * ncb
    nworkers = 32
    cpw = nchunks // nworkers
    mesh = plsc.VectorSubcoreMesh(core_axis_name="c", subcore_axis_name="s")

    @functools.partial(
        pl.kernel,
        out_type=jax.ShapeDtypeStruct((nb, n, m), jnp.float32),
        mesh=mesh,
        scratch_types=[
            pltpu.VMEM((_HALF, 16), jnp.float32),
            pltpu.VMEM((_HALF, 16), jnp.float32),
            pltpu.VMEM((_RADIX * 16,), jnp.int32),
        ],
        compiler_params=pltpu.CompilerParams(use_tc_tiling_on_sc=False, needs_layout_passes=False),
    )
    def sc_sort(x_hbm, z_hbm, o_hbm, buf_a, buf_b, table):
        lane = lax.iota(jnp.int32, 16)
        ones = jnp.ones((16,), jnp.int32)
        wid = lax.axis_index("s") * 2 + lax.axis_index("c")

        def chunk_body(ci, _):
            chunk = wid * cpw + ci
            bb = chunk // (2 * ncb)
            rem = chunk % (2 * ncb)
            hh = rem // ncb
            cb = rem % ncb
            row0 = hh * _HALF
            col0 = cb * 16
            pltpu.sync_copy(x_hbm.at[bb, pl.ds(row0, _HALF), pl.ds(col0, 16)], buf_a)
            m2 = jnp.where(hh == 1, jnp.int32(-1), jnp.int32(0))

            for p, sh in enumerate(_DIG_SHIFTS):
                src = buf_a if p % 2 == 0 else buf_b
                dst = buf_b if p % 2 == 0 else buf_a
                last = p == len(_DIG_SHIFTS) - 1

                pltpu.sync_copy(z_hbm, table)

                if p == 0:
                    def hist_body(r, _, sh=sh):
                        v = lax.bitcast_convert_type(buf_a[r], jnp.int32)
                        s = lax.shift_right_arithmetic(v, 31)
                        k = v ^ (s | _I32_MIN) ^ m2
                        buf_a[r] = lax.bitcast_convert_type(k, jnp.float32)
                        dig = lax.shift_right_logical(k, sh) & (_RADIX - 1)
                        plsc.addupdate_scatter(table, [(dig << 4) | lane], ones)
                        return 0
                else:
                    def hist_body(r, _, src=src, sh=sh):
                        k = lax.bitcast_convert_type(src[r], jnp.int32)
                        dig = lax.shift_right_logical(k, sh) & (_RADIX - 1)
                        plsc.addupdate_scatter(table, [(dig << 4) | lane], ones)
                        return 0

                lax.fori_loop(0, _HALF, hist_body, 0, unroll=4)

                def scan_body(g, acc):
                    h = table[pl.ds(g * 16, 16)]
                    table[pl.ds(g * 16, 16)] = acc
                    return acc + h

                lax.fori_loop(0, _RADIX, scan_body, jnp.zeros((16,), jnp.int32),
                              unroll=8)

                def perm_body(r, _, src=src, dst=dst, sh=sh, last=last):
                    v = src[r]
                    k = lax.bitcast_convert_type(v, jnp.int32)
                    dig = lax.shift_right_logical(k, sh) & (_RADIX - 1)
                    idx = (dig << 4) | lane
                    rank = plsc.load_gather(table, [idx])
                    plsc.store_scatter(table, [idx], rank + 1)
                    if last:
                        t = k ^ m2
                        st = lax.shift_right_arithmetic(t, 31)
                        v = lax.bitcast_convert_type(t ^ ((~st) | _I32_MIN), jnp.float32)
                    plsc.store_scatter(dst, [rank, lane], v)
                    return 0

                lax.fori_loop(0, _HALF, perm_body, 0, unroll=2)

            pltpu.sync_copy(buf_b, o_hbm.at[bb, pl.ds(row0, _HALF), pl.ds(col0, 16)])
            return 0

        lax.fori_loop(0, cpw, chunk_body, 0)

    zeros = jnp.zeros((_RADIX * 16,), jnp.int32)
    return sc_sort(x, zeros)


def _ce_reshape(x, k, d):
    n, lanes = x.shape
    g = n // (2 * d)
    x4 = x.reshape(g, 2, d, lanes)
    lo = x4[:, 0]
    hi = x4[:, 1]
    mn = jnp.minimum(lo, hi)
    mx = jnp.maximum(lo, hi)
    obit = (jax.lax.broadcasted_iota(jnp.int32, (g, 1, 1), 0) * (2 * d)) >> (k + 1)
    desc = (obit & 1) == 1
    new_lo = jnp.where(desc, mx, mn)
    new_hi = jnp.where(desc, mn, mx)
    return jnp.concatenate(
        [new_lo.reshape(g, 1, d, lanes), new_hi.reshape(g, 1, d, lanes)], axis=1
    ).reshape(n, lanes)


def _ce_roll(x, k, d):
    n, lanes = x.shape
    i = jax.lax.broadcasted_iota(jnp.int32, (n, 1), 0)
    up = pltpu.roll(x, n - d, axis=0)
    down = pltpu.roll(x, d, axis=0)
    low_half = (i & d) == 0
    partner = jnp.where(low_half, up, down)
    desc = (i >> (k + 1)) & 1 == 1
    keep_min = low_half != desc
    return jnp.where(keep_min, jnp.minimum(x, partner), jnp.maximum(x, partner))


def _merge_body(x_ref, o_ref):
    x = x_ref[0]
    n = x.shape[0]
    log_n = n.bit_length() - 1
    for j in range(log_n - 1, -1, -1):
        d = 1 << j
        if d >= 8:
            x = _ce_reshape(x, log_n - 1, d)
        else:
            x = _ce_roll(x, log_n - 1, d)
    o_ref[0] = x


def _sort_body(x_ref, o_ref):
    x = x_ref[0]
    n = x.shape[0]
    log_n = n.bit_length() - 1
    for k in range(log_n):
        for j in range(k, -1, -1):
            d = 1 << j
            if d >= 8:
                x = _ce_reshape(x, k, d)
            else:
                x = _ce_roll(x, k, d)
    o_ref[0] = x


_SC_LANES = 512
_BL = 256
_SC_BLOCKS = _SC_LANES // _BL


def _tc_sort_high(x):
    b, n, m = x.shape
    grid = (b, m // _BL - _SC_BLOCKS)
    spec = pl.BlockSpec((1, n, _BL), lambda i, j: (i, 0, j + _SC_BLOCKS))
    return pl.pallas_call(
        _sort_body,
        grid=grid,
        in_specs=[spec],
        out_specs=spec,
        out_shape=jax.ShapeDtypeStruct(x.shape, x.dtype),
    )(x)


def _tc_merge_low(y_half, y_full):
    b, n, m = y_full.shape
    grid = (b, _SC_BLOCKS)
    spec = pl.BlockSpec((1, n, _BL), lambda i, j: (i, 0, j))

    def body(half_ref, full_ref, o_ref):
        del full_ref
        _merge_body(half_ref, o_ref)

    return pl.pallas_call(
        body,
        grid=grid,
        in_specs=[spec, spec],
        out_specs=spec,
        out_shape=jax.ShapeDtypeStruct(y_full.shape, y_full.dtype),
        input_output_aliases={1: 0},
    )(y_half, y_full)


@jax.jit
def kernel(x):
    y_half = _sc_half_sort(x, _SC_LANES)
    y_full = _tc_sort_high(x)
    return _tc_merge_low(y_half, y_full)


# speedup vs baseline: 1.0189x; 1.0189x over previous
import functools

import jax
import jax.numpy as jnp
from jax import lax
from jax.experimental import pallas as pl
from jax.experimental.pallas import tpu as pltpu
from jax.experimental.pallas import tpu_sc as plsc

_HALF = 2048
_RADIX = 2048
_DIG_SHIFTS = (0, 11, 22)
_I32_MIN = -2147483648


def _sc_half_sort(x, m):
    nb, n, m_full = x.shape
    ncb = m // 16
    nchunks = nb * 2 * ncb
    nworkers = 32
    cpw = nchunks // nworkers
    mesh = plsc.VectorSubcoreMesh(core_axis_name="c", subcore_axis_name="s")

    @functools.partial(
        pl.kernel,
        out_type=jax.ShapeDtypeStruct((nb, n, m), jnp.float32),
        mesh=mesh,
        scratch_types=[
            pltpu.VMEM((_HALF, 16), jnp.float32),
            pltpu.VMEM((_HALF, 16), jnp.float32),
            pltpu.VMEM((_RADIX * 16,), jnp.int32),
        ],
        compiler_params=pltpu.CompilerParams(use_tc_tiling_on_sc=False, needs_layout_passes=False),
    )
    def sc_sort(x_hbm, o_hbm, buf_a, buf_b, table):
        lane = lax.iota(jnp.int32, 16)
        ones = jnp.ones((16,), jnp.int32)
        wid = lax.axis_index("s") * 2 + lax.axis_index("c")

        def chunk_body(ci, _):
            chunk = wid * cpw + ci
            bb = chunk // (2 * ncb)
            rem = chunk % (2 * ncb)
            hh = rem // ncb
            cb = rem % ncb
            row0 = hh * _HALF
            col0 = cb * 16
            pltpu.sync_copy(x_hbm.at[bb, pl.ds(row0, _HALF), pl.ds(col0, 16)], buf_a)
            m2 = jnp.where(hh == 1, jnp.int32(-1), jnp.int32(0))

            for p, sh in enumerate(_DIG_SHIFTS):
                src = buf_a if p % 2 == 0 else buf_b
                dst = buf_b if p % 2 == 0 else buf_a
                last = p == len(_DIG_SHIFTS) - 1

                def zero_body(i, _):
                    table[pl.ds(i * 16, 16)] = jnp.zeros((16,), jnp.int32)
                    return 0

                lax.fori_loop(0, _RADIX, zero_body, 0, unroll=8)

                if p == 0:
                    def hist_body(r, _, sh=sh):
                        v = lax.bitcast_convert_type(buf_a[r], jnp.int32)
                        s = lax.shift_right_arithmetic(v, 31)
                        k = v ^ (s | _I32_MIN) ^ m2
                        buf_a[r] = lax.bitcast_convert_type(k, jnp.float32)
                        dig = lax.shift_right_logical(k, sh) & (_RADIX - 1)
                        plsc.addupdate_scatter(table, [(dig << 4) | lane], ones)
                        return 0
                else:
                    def hist_body(r, _, src=src, sh=sh):
                        k = lax.bitcast_convert_type(src[r], jnp.int32)
                        dig = lax.shift_right_logical(k, sh) & (_RADIX - 1)
                        plsc.addupdate_scatter(table, [(dig << 4) | lane], ones)
                        return 0

                lax.fori_loop(0, _HALF, hist_body, 0, unroll=4)

                def scan_body(g, acc):
                    h = table[pl.ds(g * 16, 16)]
                    table[pl.ds(g * 16, 16)] = acc
                    return acc + h

                lax.fori_loop(0, _RADIX, scan_body, jnp.zeros((16,), jnp.int32),
                              unroll=8)

                def perm_body(r, _, src=src, dst=dst, sh=sh, last=last):
                    v = src[r]
                    k = lax.bitcast_convert_type(v, jnp.int32)
                    dig = lax.shift_right_logical(k, sh) & (_RADIX - 1)
                    idx = (dig << 4) | lane
                    rank = plsc.load_gather(table, [idx])
                    plsc.store_scatter(table, [idx], rank + 1)
                    if last:
                        t = k ^ m2
                        st = lax.shift_right_arithmetic(t, 31)
                        v = lax.bitcast_convert_type(t ^ ((~st) | _I32_MIN), jnp.float32)
                    plsc.store_scatter(dst, [rank, lane], v)
                    return 0

                lax.fori_loop(0, _HALF, perm_body, 0, unroll=2)

            pltpu.sync_copy(buf_b, o_hbm.at[bb, pl.ds(row0, _HALF), pl.ds(col0, 16)])
            return 0

        lax.fori_loop(0, cpw, chunk_body, 0)

    return sc_sort(x)


def _ce_reshape(x, k, d):
    n, lanes = x.shape
    g = n // (2 * d)
    x4 = x.reshape(g, 2, d, lanes)
    lo = x4[:, 0]
    hi = x4[:, 1]
    mn = jnp.minimum(lo, hi)
    mx = jnp.maximum(lo, hi)
    obit = (jax.lax.broadcasted_iota(jnp.int32, (g, 1, 1), 0) * (2 * d)) >> (k + 1)
    desc = (obit & 1) == 1
    new_lo = jnp.where(desc, mx, mn)
    new_hi = jnp.where(desc, mn, mx)
    return jnp.concatenate(
        [new_lo.reshape(g, 1, d, lanes), new_hi.reshape(g, 1, d, lanes)], axis=1
    ).reshape(n, lanes)


def _ce_roll(x, k, d):
    n, lanes = x.shape
    i = jax.lax.broadcasted_iota(jnp.int32, (n, 1), 0)
    up = pltpu.roll(x, n - d, axis=0)
    down = pltpu.roll(x, d, axis=0)
    low_half = (i & d) == 0
    partner = jnp.where(low_half, up, down)
    desc = (i >> (k + 1)) & 1 == 1
    keep_min = low_half != desc
    return jnp.where(keep_min, jnp.minimum(x, partner), jnp.maximum(x, partner))


def _merge_body(x_ref, o_ref):
    x = x_ref[0]
    n = x.shape[0]
    log_n = n.bit_length() - 1
    for j in range(log_n - 1, -1, -1):
        d = 1 << j
        if d >= 8:
            x = _ce_reshape(x, log_n - 1, d)
        else:
            x = _ce_roll(x, log_n - 1, d)
    o_ref[0] = x


def _sort_body(x_ref, o_ref):
    x = x_ref[0]
    n = x.shape[0]
    log_n = n.bit_length() - 1
    for k in range(log_n):
        for j in range(k, -1, -1):
            d = 1 << j
            if d >= 8:
                x = _ce_reshape(x, k, d)
            else:
                x = _ce_roll(x, k, d)
    o_ref[0] = x


_SC_LANES = 512
_BL = 256
_SC_BLOCKS = _SC_LANES // _BL


def _tc_sort_high(x):
    b, n, m = x.shape
    grid = (b, m // _BL - _SC_BLOCKS)
    spec = pl.BlockSpec((1, n, _BL), lambda i, j: (i, 0, j + _SC_BLOCKS))
    return pl.pallas_call(
        _sort_body,
        grid=grid,
        in_specs=[spec],
        out_specs=spec,
        out_shape=jax.ShapeDtypeStruct(x.shape, x.dtype),
    )(x)


def _tc_merge_low(y_half, y_full):
    b, n, m = y_full.shape
    grid = (b, _SC_BLOCKS)
    spec = pl.BlockSpec((1, n, _BL), lambda i, j: (i, 0, j))

    def body(half_ref, full_ref, o_ref):
        del full_ref
        _merge_body(half_ref, o_ref)

    return pl.pallas_call(
        body,
        grid=grid,
        in_specs=[spec, spec],
        out_specs=spec,
        out_shape=jax.ShapeDtypeStruct(y_full.shape, y_full.dtype),
        input_output_aliases={1: 0},
    )(y_half, y_full)


@jax.jit
def kernel(x):
    y_half = _sc_half_sort(x, _SC_LANES)
    y_full = _tc_sort_high(x)
    return _tc_merge_low(y_half, y_full)


# speedup vs baseline: 1.0804x; 1.0603x over previous
import functools

import jax
import jax.numpy as jnp
from jax import lax
from jax.experimental import pallas as pl
from jax.experimental.pallas import tpu as pltpu
from jax.experimental.pallas import tpu_sc as plsc

_HALF = 2048
_RADIX = 2048
_DIG_SHIFTS = (0, 11, 22)
_I32_MIN = -2147483648


def _sc_half_sort(x, m):
    nb, n, m_full = x.shape
    ncb = m // 16
    nchunks = nb * 2 * ncb
    nworkers = 32
    cpw = nchunks // nworkers
    mesh = plsc.VectorSubcoreMesh(core_axis_name="c", subcore_axis_name="s")

    @functools.partial(
        pl.kernel,
        out_type=jax.ShapeDtypeStruct((nb, n, m), jnp.float32),
        mesh=mesh,
        scratch_types=[
            pltpu.VMEM((_HALF, 16), jnp.float32),
            pltpu.VMEM((_HALF, 16), jnp.float32),
            pltpu.VMEM((_RADIX * 16,), jnp.int32),
        ],
        compiler_params=pltpu.CompilerParams(use_tc_tiling_on_sc=False, needs_layout_passes=False),
    )
    def sc_sort(x_hbm, o_hbm, buf_a, buf_b, table):
        lane = lax.iota(jnp.int32, 16)
        ones = jnp.ones((16,), jnp.int32)
        wid = lax.axis_index("s") * 2 + lax.axis_index("c")

        def chunk_body(ci, _):
            chunk = wid * cpw + ci
            bb = chunk // (2 * ncb)
            rem = chunk % (2 * ncb)
            hh = rem // ncb
            cb = rem % ncb
            row0 = hh * _HALF
            col0 = cb * 16
            pltpu.sync_copy(x_hbm.at[bb, pl.ds(row0, _HALF), pl.ds(col0, 16)], buf_a)
            m2 = jnp.where(hh == 1, jnp.int32(-1), jnp.int32(0))

            for p, sh in enumerate(_DIG_SHIFTS):
                src = buf_a if p % 2 == 0 else buf_b
                dst = buf_b if p % 2 == 0 else buf_a
                last = p == len(_DIG_SHIFTS) - 1

                def zero_body(i, _):
                    table[pl.ds(i * 16, 16)] = jnp.zeros((16,), jnp.int32)
                    return 0

                lax.fori_loop(0, _RADIX, zero_body, 0, unroll=8)

                if p == 0:
                    def hist_body(r, _, sh=sh):
                        v = lax.bitcast_convert_type(buf_a[r], jnp.int32)
                        s = lax.shift_right_arithmetic(v, 31)
                        k = v ^ (s | _I32_MIN) ^ m2
                        buf_a[r] = lax.bitcast_convert_type(k, jnp.float32)
                        dig = lax.shift_right_logical(k, sh) & (_RADIX - 1)
                        plsc.addupdate_scatter(table, [(dig << 4) | lane], ones)
                        return 0
                else:
                    def hist_body(r, _, src=src, sh=sh):
                        k = lax.bitcast_convert_type(src[r], jnp.int32)
                        dig = lax.shift_right_logical(k, sh) & (_RADIX - 1)
                        plsc.addupdate_scatter(table, [(dig << 4) | lane], ones)
                        return 0

                lax.fori_loop(0, _HALF, hist_body, 0, unroll=4)

                def scan_body(g, acc):
                    h = table[pl.ds(g * 16, 16)]
                    table[pl.ds(g * 16, 16)] = acc
                    return acc + h

                lax.fori_loop(0, _RADIX, scan_body, jnp.zeros((16,), jnp.int32),
                              unroll=8)

                def perm_body(r, _, src=src, dst=dst, sh=sh, last=last):
                    v = src[r]
                    k = lax.bitcast_convert_type(v, jnp.int32)
                    dig = lax.shift_right_logical(k, sh) & (_RADIX - 1)
                    idx = (dig << 4) | lane
                    rank = plsc.load_gather(table, [idx])
                    plsc.store_scatter(table, [idx], rank + 1)
                    if last:
                        t = k ^ m2
                        st = lax.shift_right_arithmetic(t, 31)
                        v = lax.bitcast_convert_type(t ^ ((~st) | _I32_MIN), jnp.float32)
                    plsc.store_scatter(dst, [rank, lane], v)
                    return 0

                lax.fori_loop(0, _HALF, perm_body, 0, unroll=2)

            pltpu.sync_copy(buf_b, o_hbm.at[bb, pl.ds(row0, _HALF), pl.ds(col0, 16)])
            return 0

        lax.fori_loop(0, cpw, chunk_body, 0)

    return sc_sort(x)


def _ce_reshape(x, k, d):
    n, lanes = x.shape
    g = n // (2 * d)
    x4 = x.reshape(g, 2, d, lanes)
    lo = x4[:, 0]
    hi = x4[:, 1]
    mn = jnp.minimum(lo, hi)
    mx = jnp.maximum(lo, hi)
    obit = (jax.lax.broadcasted_iota(jnp.int32, (g, 1, 1), 0) * (2 * d)) >> (k + 1)
    desc = (obit & 1) == 1
    new_lo = jnp.where(desc, mx, mn)
    new_hi = jnp.where(desc, mn, mx)
    return jnp.concatenate(
        [new_lo.reshape(g, 1, d, lanes), new_hi.reshape(g, 1, d, lanes)], axis=1
    ).reshape(n, lanes)


def _ce_roll(x, k, d):
    n, lanes = x.shape
    i = jax.lax.broadcasted_iota(jnp.int32, (n, 1), 0)
    up = pltpu.roll(x, n - d, axis=0)
    down = pltpu.roll(x, d, axis=0)
    low_half = (i & d) == 0
    partner = jnp.where(low_half, up, down)
    desc = (i >> (k + 1)) & 1 == 1
    keep_min = low_half != desc
    return jnp.where(keep_min, jnp.minimum(x, partner), jnp.maximum(x, partner))


def _merge_body(x_ref, o_ref):
    x = x_ref[0]
    n = x.shape[0]
    log_n = n.bit_length() - 1
    for j in range(log_n - 1, -1, -1):
        d = 1 << j
        if d >= 8:
            x = _ce_reshape(x, log_n - 1, d)
        else:
            x = _ce_roll(x, log_n - 1, d)
    o_ref[0] = x


def _sort_body(x_ref, o_ref):
    x = x_ref[0]
    n = x.shape[0]
    log_n = n.bit_length() - 1
    for k in range(log_n):
        for j in range(k, -1, -1):
            d = 1 << j
            if d >= 8:
                x = _ce_reshape(x, k, d)
            else:
                x = _ce_roll(x, k, d)
    o_ref[0] = x


_SC_LANES = 384


def _tc_sort_slab(x, y_prev, lane0, bl, nblocks):
    b, n, m = x.shape
    grid = (b, nblocks)
    spec = pl.BlockSpec((1, n, bl), lambda i, j, o=lane0 // bl: (i, 0, j + o))
    out_shape = jax.ShapeDtypeStruct(x.shape, x.dtype)
    if y_prev is None:
        return pl.pallas_call(
            _sort_body,
            grid=grid,
            in_specs=[spec],
            out_specs=spec,
            out_shape=out_shape,
        )(x)

    def body(x_ref, prev_ref, o_ref):
        del prev_ref
        _sort_body(x_ref, o_ref)

    return pl.pallas_call(
        body,
        grid=grid,
        in_specs=[spec, spec],
        out_specs=spec,
        out_shape=out_shape,
        input_output_aliases={1: 0},
    )(x, y_prev)


def _tc_merge_low(y_half, y_full):
    b, n, m = y_full.shape
    bl = 128
    grid = (b, _SC_LANES // bl)
    spec = pl.BlockSpec((1, n, bl), lambda i, j: (i, 0, j))

    def body(half_ref, full_ref, o_ref):
        del full_ref
        _merge_body(half_ref, o_ref)

    return pl.pallas_call(
        body,
        grid=grid,
        in_specs=[spec, spec],
        out_specs=spec,
        out_shape=jax.ShapeDtypeStruct(y_full.shape, y_full.dtype),
        input_output_aliases={1: 0},
    )(y_half, y_full)


@jax.jit
def kernel(x):
    y_half = _sc_half_sort(x, _SC_LANES)
    y = _tc_sort_slab(x, None, 512, 256, 2)
    y = _tc_sort_slab(x, y, _SC_LANES, 128, 1)
    return _tc_merge_low(y_half, y)


# speedup vs baseline: 1.0830x; 1.0025x over previous
import functools

import jax
import jax.numpy as jnp
from jax import lax
from jax.experimental import pallas as pl
from jax.experimental.pallas import tpu as pltpu
from jax.experimental.pallas import tpu_sc as plsc

_HALF = 2048
_RADIX = 2048
_DIG_SHIFTS = (0, 11, 22)
_I32_MIN = -2147483648


def _sc_half_sort(x, m):
    nb, n, m_full = x.shape
    ncb = m // 16
    nchunks = nb * 2 * ncb
    nworkers = 32
    cpw = nchunks // nworkers
    mesh = plsc.VectorSubcoreMesh(core_axis_name="c", subcore_axis_name="s")

    @functools.partial(
        pl.kernel,
        out_type=jax.ShapeDtypeStruct((nb, n, m), jnp.float32),
        mesh=mesh,
        scratch_types=[
            pltpu.VMEM((_HALF, 16), jnp.float32),
            pltpu.VMEM((_HALF, 16), jnp.float32),
            pltpu.VMEM((_RADIX * 16,), jnp.int32),
        ],
        compiler_params=pltpu.CompilerParams(use_tc_tiling_on_sc=False, needs_layout_passes=False),
    )
    def sc_sort(x_hbm, o_hbm, buf_a, buf_b, table):
        lane = lax.iota(jnp.int32, 16)
        ones = jnp.ones((16,), jnp.int32)
        wid = lax.axis_index("s") * 2 + lax.axis_index("c")

        def chunk_body(ci, _):
            chunk = wid * cpw + ci
            bb = chunk // (2 * ncb)
            rem = chunk % (2 * ncb)
            hh = rem // ncb
            cb = rem % ncb
            row0 = hh * _HALF
            col0 = cb * 16
            pltpu.sync_copy(x_hbm.at[bb, pl.ds(row0, _HALF), pl.ds(col0, 16)], buf_a)
            m2 = jnp.where(hh == 1, jnp.int32(-1), jnp.int32(0))

            for p, sh in enumerate(_DIG_SHIFTS):
                src = buf_a if p % 2 == 0 else buf_b
                dst = buf_b if p % 2 == 0 else buf_a
                last = p == len(_DIG_SHIFTS) - 1

                def zero_body(i, _):
                    table[pl.ds(i * 16, 16)] = jnp.zeros((16,), jnp.int32)
                    return 0

                lax.fori_loop(0, _RADIX, zero_body, 0, unroll=8)

                if p == 0:
                    def hist_body(r, _, sh=sh):
                        v = lax.bitcast_convert_type(buf_a[r], jnp.int32)
                        s = lax.shift_right_arithmetic(v, 31)
                        k = v ^ (s | _I32_MIN) ^ m2
                        buf_a[r] = lax.bitcast_convert_type(k, jnp.float32)
                        dig = lax.shift_right_logical(k, sh) & (_RADIX - 1)
                        plsc.addupdate_scatter(table, [(dig << 4) | lane], ones)
                        return 0
                else:
                    def hist_body(r, _, src=src, sh=sh):
                        k = lax.bitcast_convert_type(src[r], jnp.int32)
                        dig = lax.shift_right_logical(k, sh) & (_RADIX - 1)
                        plsc.addupdate_scatter(table, [(dig << 4) | lane], ones)
                        return 0

                lax.fori_loop(0, _HALF, hist_body, 0, unroll=4)

                def scan_body(g, acc):
                    h = table[pl.ds(g * 16, 16)]
                    table[pl.ds(g * 16, 16)] = acc
                    return acc + h

                lax.fori_loop(0, _RADIX, scan_body, jnp.zeros((16,), jnp.int32),
                              unroll=8)

                def perm_body(r, _, src=src, dst=dst, sh=sh, last=last):
                    v = src[r]
                    k = lax.bitcast_convert_type(v, jnp.int32)
                    dig = lax.shift_right_logical(k, sh) & (_RADIX - 1)
                    idx = (dig << 4) | lane
                    rank = plsc.load_gather(table, [idx])
                    plsc.store_scatter(table, [idx], rank + 1)
                    if last:
                        t = k ^ m2
                        st = lax.shift_right_arithmetic(t, 31)
                        v = lax.bitcast_convert_type(t ^ ((~st) | _I32_MIN), jnp.float32)
                    plsc.store_scatter(dst, [rank, lane], v)
                    return 0

                lax.fori_loop(0, _HALF, perm_body, 0, unroll=2)

            pltpu.sync_copy(buf_b, o_hbm.at[bb, pl.ds(row0, _HALF), pl.ds(col0, 16)])
            return 0

        lax.fori_loop(0, cpw, chunk_body, 0)

    return sc_sort(x)


def _ce_reshape(x, k, d):
    n, lanes = x.shape
    g = n // (2 * d)
    x4 = x.reshape(g, 2, d, lanes)
    lo = x4[:, 0]
    hi = x4[:, 1]
    mn = jnp.minimum(lo, hi)
    mx = jnp.maximum(lo, hi)
    obit = (jax.lax.broadcasted_iota(jnp.int32, (g, 1, 1), 0) * (2 * d)) >> (k + 1)
    desc = (obit & 1) == 1
    new_lo = jnp.where(desc, mx, mn)
    new_hi = jnp.where(desc, mn, mx)
    return jnp.concatenate(
        [new_lo.reshape(g, 1, d, lanes), new_hi.reshape(g, 1, d, lanes)], axis=1
    ).reshape(n, lanes)


def _ce_roll(x, k, d):
    n, lanes = x.shape
    i = jax.lax.broadcasted_iota(jnp.int32, (n, 1), 0)
    up = pltpu.roll(x, n - d, axis=0)
    down = pltpu.roll(x, d, axis=0)
    low_half = (i & d) == 0
    partner = jnp.where(low_half, up, down)
    desc = (i >> (k + 1)) & 1 == 1
    keep_min = low_half != desc
    return jnp.where(keep_min, jnp.minimum(x, partner), jnp.maximum(x, partner))


def _merge_body(x_ref, o_ref):
    x = x_ref[0]
    n = x.shape[0]
    log_n = n.bit_length() - 1
    for j in range(log_n - 1, -1, -1):
        d = 1 << j
        if d >= 8:
            x = _ce_reshape(x, log_n - 1, d)
        else:
            x = _ce_roll(x, log_n - 1, d)
    o_ref[0] = x


def _sort_body(x_ref, o_ref):
    x = x_ref[0]
    n = x.shape[0]
    log_n = n.bit_length() - 1
    for k in range(log_n):
        for j in range(k, -1, -1):
            d = 1 << j
            if d >= 8:
                x = _ce_reshape(x, k, d)
            else:
                x = _ce_roll(x, k, d)
    o_ref[0] = x


_SC_LANES = 384


def _tc_sort_slab(x, y_prev, lane0, bl, nblocks):
    b, n, m = x.shape
    grid = (b, nblocks)
    spec = pl.BlockSpec((1, n, bl), lambda i, j, o=lane0 // bl: (i, 0, j + o))
    out_shape = jax.ShapeDtypeStruct(x.shape, x.dtype)
    if y_prev is None:
        return pl.pallas_call(
            _sort_body,
            grid=grid,
            in_specs=[spec],
            out_specs=spec,
            out_shape=out_shape,
        )(x)

    def body(x_ref, prev_ref, o_ref):
        del prev_ref
        _sort_body(x_ref, o_ref)

    tiny = pl.BlockSpec((1, 8, 128), lambda i, j: (0, 0, 0))
    return pl.pallas_call(
        body,
        grid=grid,
        in_specs=[spec, tiny],
        out_specs=spec,
        out_shape=out_shape,
        input_output_aliases={1: 0},
    )(x, y_prev)


def _tc_merge_low(y_half, y_full):
    b, n, m = y_full.shape
    bl = _SC_LANES
    grid = (b, 1)
    spec = pl.BlockSpec((1, n, bl), lambda i, j: (i, 0, j))

    def body(half_ref, full_ref, o_ref):
        del full_ref
        _merge_body(half_ref, o_ref)

    tiny = pl.BlockSpec((1, 8, 128), lambda i, j: (0, 0, 0))
    return pl.pallas_call(
        body,
        grid=grid,
        in_specs=[spec, tiny],
        out_specs=spec,
        out_shape=jax.ShapeDtypeStruct(y_full.shape, y_full.dtype),
        input_output_aliases={1: 0},
    )(y_half, y_full)


@jax.jit
def kernel(x):
    y_half = _sc_half_sort(x, _SC_LANES)
    y = _tc_sort_slab(x, None, 512, 256, 2)
    y = _tc_sort_slab(x, y, _SC_LANES, 128, 1)
    return _tc_merge_low(y_half, y)
